# Initial kernel scaffold; baseline (speedup 1.0000x reference)
import jax
import jax.numpy as jnp
from jax.experimental import pallas as pl


def kernel(x, W_enc, W_dec, b_enc, b_dec):
    raise NotImplementedError("write your pallas kernel here")


# speedup vs baseline: 9.2106x; 9.2106x over previous
import jax
import jax.numpy as jnp
from jax.experimental import pallas as pl

B = 4096
D_IN = 2048
D_SAE = 16384
K = 64
EPS = 1e-5

_INT_MIN = -2147483648
_INT_MAX = 2147483647
_MANT = 0x7FFFFFFF


def _layernorm(x):
    mu = jnp.mean(x, axis=1, keepdims=True)
    xc = x - mu
    var = jnp.sum(xc * xc, axis=1, keepdims=True) / (D_IN - 1)
    return xc / (jnp.sqrt(var) + EPS)


def _encode_kernel(x_ref, bdec_ref, w_ref, benc_ref, out_ref):
    xp = _layernorm(x_ref[...])
    xin = (xp - bdec_ref[...]).astype(jnp.bfloat16)
    out_ref[...] = (
        jnp.dot(xin, w_ref[...].astype(jnp.bfloat16),
                preferred_element_type=jnp.float32)
        + benc_ref[...]
    )


def _encode(x, W_enc, b_enc, b_dec, bm, bn):
    ni, nj = B // bm, D_SAE // bn
    return pl.pallas_call(
        _encode_kernel,
        grid=(nj, ni),
        in_specs=[
            pl.BlockSpec((bm, D_IN), lambda j, i: (i, 0)),
            pl.BlockSpec((1, D_IN), lambda j, i: (0, 0)),
            pl.BlockSpec((D_IN, bn), lambda j, i: (0, j)),
            pl.BlockSpec((1, bn), lambda j, i: (0, j)),
        ],
        out_specs=pl.BlockSpec((bm, bn), lambda j, i: (i, j)),
        out_shape=jax.ShapeDtypeStruct((B, D_SAE), jnp.float32),
    )(x, b_dec.reshape(1, D_IN), W_enc, b_enc.reshape(1, D_SAE))


def _select_kernel(pre_ref, tau_ref):
    pre = pre_ref[...]
    bits = jax.lax.bitcast_convert_type(pre, jnp.int32)
    key = jnp.where(bits >= 0, bits, bits ^ jnp.int32(_MANT))
    cpos = jnp.sum((key >= 0).astype(jnp.int32), axis=1, keepdims=True)
    neg = cpos < K
    lo = jnp.where(neg, jnp.int32(_INT_MIN), jnp.int32(0))
    hi = jnp.where(neg, jnp.int32(-1), jnp.int32(_INT_MAX))

    def body(_, carry):
        lo, hi = carry
        span = hi - lo
        mid = lo + (span >> 1) + (span & 1)
        cnt = jnp.sum((key >= mid).astype(jnp.int32), axis=1, keepdims=True)
        ge = cnt >= K
        return jnp.where(ge, mid, lo), jnp.where(ge, hi, mid - 1)

    lo, _ = jax.lax.fori_loop(0, 31, body, (lo, hi))
    tau_bits = jnp.where(lo >= 0, lo, lo ^ jnp.int32(_MANT))
    tau = jax.lax.bitcast_convert_type(tau_bits, jnp.float32)
    tau_ref[...] = jnp.broadcast_to(tau, tau_ref.shape)


def _select(pre, bm):
    ni = B // bm
    return pl.pallas_call(
        _select_kernel,
        grid=(ni,),
        in_specs=[pl.BlockSpec((bm, D_SAE), lambda i: (i, 0))],
        out_specs=pl.BlockSpec((bm, 128), lambda i: (i, 0)),
        out_shape=jax.ShapeDtypeStruct((B, 128), jnp.float32),
    )(pre)


def _decode_kernel(pre_ref, tau_ref, w_ref, x_ref, bdec_ref,
                   out_ref, s_ref, col_ref):
    i = pl.program_id(0)
    k = pl.program_id(1)
    nk = pl.num_programs(1)
    pre = pre_ref[...]
    tau = tau_ref[:, 0:1]
    lat = jnp.where(pre >= tau, jnp.maximum(pre, 0.0), 0.0)
    contrib = jnp.dot(lat.astype(jnp.bfloat16), w_ref[...].astype(jnp.bfloat16),
                      preferred_element_type=jnp.float32)

    @pl.when(k == 0)
    def _():
        out_ref[...] = contrib + bdec_ref[...]

    @pl.when(k > 0)
    def _():
        out_ref[...] += contrib

    @pl.when(jnp.logical_and(i == 0, k == 0))
    def _():
        s_ref[...] = jnp.zeros_like(s_ref)
        col_ref[...] = jnp.zeros_like(col_ref)

    @pl.when(k == nk - 1)
    def _():
        xp = _layernorm(x_ref[...])
        diff = out_ref[...] - xp
        s0 = jnp.sum(diff * diff)
        s1 = jnp.sum(xp * xp)
        row = jax.lax.broadcasted_iota(jnp.int32, s_ref.shape, 0)
        s_ref[...] += jnp.where(row == 0, s0, s1) * (row < 2)
        col_ref[...] += jnp.broadcast_to(
            jnp.sum(xp, axis=0, keepdims=True), col_ref.shape)


def _decode(pre, tau, W_dec, x, b_dec, bm, bk):
    ni, nk = B // bm, D_SAE // bk
    return pl.pallas_call(
        _decode_kernel,
        grid=(ni, nk),
        in_specs=[
            pl.BlockSpec((bm, bk), lambda i, k: (i, k)),
            pl.BlockSpec((bm, 128), lambda i, k: (i, 0)),
            pl.BlockSpec((bk, D_IN), lambda i, k: (k, 0)),
            pl.BlockSpec((bm, D_IN), lambda i, k: (i, 0)),
            pl.BlockSpec((1, D_IN), lambda i, k: (0, 0)),
        ],
        out_specs=[
            pl.BlockSpec((bm, D_IN), lambda i, k: (i, 0)),
            pl.BlockSpec((8, 128), lambda i, k: (0, 0)),
            pl.BlockSpec((8, D_IN), lambda i, k: (0, 0)),
        ],
        out_shape=[
            jax.ShapeDtypeStruct((B, D_IN), jnp.float32),
            jax.ShapeDtypeStruct((8, 128), jnp.float32),
            jax.ShapeDtypeStruct((8, D_IN), jnp.float32),
        ],
    )(pre, tau, W_dec, x, b_dec.reshape(1, D_IN))


ENC_BM = 256
ENC_BN = 1024
SEL_BM = 128
DEC_BM = 512
DEC_BK = 512


def kernel(x, W_enc, W_dec, b_enc, b_dec):
    pre = _encode(x, W_enc, b_enc, b_dec, bm=ENC_BM, bn=ENC_BN)
    tau = _select(pre, bm=SEL_BM)
    recons, s, col = _decode(pre, tau, W_dec, x, b_dec, bm=DEC_BM, bk=DEC_BK)
    s0 = s[0, 0]
    s1 = s[1, 0]
    colsum = col[0]
    denom = jnp.float32(B * D_IN)
    mse = s0 / denom
    mse_naive = (s1 - jnp.sum(colsum * colsum) / B) / denom
    mse_loss = mse / mse_naive
    aux_loss = jnp.asarray(0.0, dtype=jnp.float32)
    loss = mse_loss + aux_loss
    return recons, loss, mse_loss, aux_loss


# speedup vs baseline: 10.1480x; 1.1018x over previous
import jax
import jax.numpy as jnp
from jax.experimental import pallas as pl

B = 4096
D_IN = 2048
D_SAE = 16384
K = 64
EPS = 1e-5

_INT_MIN = -2147483648
_INT_MAX = 2147483647
_MANT = 0x7FFFFFFF


def _layernorm(x):
    mu = jnp.mean(x, axis=1, keepdims=True)
    xc = x - mu
    var = jnp.sum(xc * xc, axis=1, keepdims=True) / (D_IN - 1)
    return xc / (jnp.sqrt(var) + EPS)


def _encode_kernel(x_ref, bdec_ref, w_ref, benc_ref, out_ref):
    xp = _layernorm(x_ref[...])
    xin = (xp - bdec_ref[...]).astype(jnp.bfloat16)
    out_ref[...] = (
        jnp.dot(xin, w_ref[...], preferred_element_type=jnp.float32)
        + benc_ref[...]
    )


def _encode(x, W_enc, b_enc, b_dec, bm, bn):
    ni, nj = B // bm, D_SAE // bn
    return pl.pallas_call(
        _encode_kernel,
        grid=(nj, ni),
        in_specs=[
            pl.BlockSpec((bm, D_IN), lambda j, i: (i, 0)),
            pl.BlockSpec((1, D_IN), lambda j, i: (0, 0)),
            pl.BlockSpec((D_IN, bn), lambda j, i: (0, j)),
            pl.BlockSpec((1, bn), lambda j, i: (0, j)),
        ],
        out_specs=pl.BlockSpec((bm, bn), lambda j, i: (i, j)),
        out_shape=jax.ShapeDtypeStruct((B, D_SAE), jnp.float32),
    )(x, b_dec.reshape(1, D_IN), W_enc, b_enc.reshape(1, D_SAE))


def _select_kernel(pre_ref, tau_ref):
    pre = pre_ref[...]
    bits = jax.lax.bitcast_convert_type(pre, jnp.int32)
    key = jnp.where(bits >= 0, bits, bits ^ jnp.int32(_MANT))
    cpos = jnp.sum((key >= 0).astype(jnp.int32), axis=1, keepdims=True)
    neg = cpos < K
    lo = jnp.where(neg, jnp.int32(_INT_MIN), jnp.int32(0))
    hi = jnp.where(neg, jnp.int32(-1), jnp.int32(_INT_MAX))

    def body(_, carry):
        lo, hi = carry
        span = hi - lo
        mid = lo + (span >> 1) + (span & 1)
        cnt = jnp.sum((key >= mid).astype(jnp.int32), axis=1, keepdims=True)
        ge = cnt >= K
        return jnp.where(ge, mid, lo), jnp.where(ge, hi, mid - 1)

    lo, _ = jax.lax.fori_loop(0, 31, body, (lo, hi))
    tau_bits = jnp.where(lo >= 0, lo, lo ^ jnp.int32(_MANT))
    tau = jax.lax.bitcast_convert_type(tau_bits, jnp.float32)
    tau_ref[...] = jnp.broadcast_to(tau, tau_ref.shape)


def _select(pre, bm):
    ni = B // bm
    return pl.pallas_call(
        _select_kernel,
        grid=(ni,),
        in_specs=[pl.BlockSpec((bm, D_SAE), lambda i: (i, 0))],
        out_specs=pl.BlockSpec((bm, 128), lambda i: (i, 0)),
        out_shape=jax.ShapeDtypeStruct((B, 128), jnp.float32),
    )(pre)


def _decode_kernel(pre_ref, tau_ref, w_ref, x_ref, bdec_ref,
                   out_ref, s_ref, col_ref):
    i = pl.program_id(0)
    k = pl.program_id(1)
    nk = pl.num_programs(1)
    pre = pre_ref[...]
    tau = tau_ref[:, 0:1]
    lat = jnp.where(pre >= tau, jnp.maximum(pre, 0.0), 0.0)
    contrib = jnp.dot(lat.astype(jnp.bfloat16), w_ref[...],
                      preferred_element_type=jnp.float32)

    @pl.when(k == 0)
    def _():
        out_ref[...] = contrib + bdec_ref[...]

    @pl.when(k > 0)
    def _():
        out_ref[...] += contrib

    @pl.when(jnp.logical_and(i == 0, k == 0))
    def _():
        s_ref[...] = jnp.zeros_like(s_ref)
        col_ref[...] = jnp.zeros_like(col_ref)

    @pl.when(k == nk - 1)
    def _():
        xp = _layernorm(x_ref[...])
        diff = out_ref[...] - xp
        s0 = jnp.sum(diff * diff)
        s1 = jnp.sum(xp * xp)
        row = jax.lax.broadcasted_iota(jnp.int32, s_ref.shape, 0)
        s_ref[...] += jnp.where(row == 0, s0, s1) * (row < 2)
        col_ref[...] += jnp.broadcast_to(
            jnp.sum(xp, axis=0, keepdims=True), col_ref.shape)


def _decode(pre, tau, W_dec, x, b_dec, bm, bk):
    ni, nk = B // bm, D_SAE // bk
    return pl.pallas_call(
        _decode_kernel,
        grid=(ni, nk),
        in_specs=[
            pl.BlockSpec((bm, bk), lambda i, k: (i, k)),
            pl.BlockSpec((bm, 128), lambda i, k: (i, 0)),
            pl.BlockSpec((bk, D_IN), lambda i, k: (k, 0)),
            pl.BlockSpec((bm, D_IN), lambda i, k: (i, 0)),
            pl.BlockSpec((1, D_IN), lambda i, k: (0, 0)),
        ],
        out_specs=[
            pl.BlockSpec((bm, D_IN), lambda i, k: (i, 0)),
            pl.BlockSpec((8, 128), lambda i, k: (0, 0)),
            pl.BlockSpec((8, D_IN), lambda i, k: (0, 0)),
        ],
        out_shape=[
            jax.ShapeDtypeStruct((B, D_IN), jnp.float32),
            jax.ShapeDtypeStruct((8, 128), jnp.float32),
            jax.ShapeDtypeStruct((8, D_IN), jnp.float32),
        ],
    )(pre, tau, W_dec, x, b_dec.reshape(1, D_IN))


ENC_BM = 256
ENC_BN = 4096
SEL_BM = 128
DEC_BM = 1024
DEC_BK = 512


def kernel(x, W_enc, W_dec, b_enc, b_dec):
    W_enc_bf = W_enc.astype(jnp.bfloat16)
    W_dec_bf = W_dec.astype(jnp.bfloat16)
    pre = _encode(x, W_enc_bf, b_enc, b_dec, bm=ENC_BM, bn=ENC_BN)
    tau = _select(pre, bm=SEL_BM)
    recons, s, col = _decode(pre, tau, W_dec_bf, x, b_dec, bm=DEC_BM, bk=DEC_BK)
    s0 = s[0, 0]
    s1 = s[1, 0]
    colsum = col[0]
    denom = jnp.float32(B * D_IN)
    mse = s0 / denom
    mse_naive = (s1 - jnp.sum(colsum * colsum) / B) / denom
    mse_loss = mse / mse_naive
    aux_loss = jnp.asarray(0.0, dtype=jnp.float32)
    loss = mse_loss + aux_loss
    return recons, loss, mse_loss, aux_loss
